# Initial kernel scaffold; baseline (speedup 1.0000x reference)
import jax
import jax.numpy as jnp
from jax.experimental import pallas as pl


def kernel(x, ref_input, qkv_w, qkv_b, attn_out_w, attn_out_b, bil_w, bil_b, bil_out_w, bil_out_b, mlp_out_w, mlp_out_b, w_ipa, w_daa, norm_mask):
    raise NotImplementedError("write your pallas kernel here")


# speedup vs baseline: 2.7416x; 2.7416x over previous
import functools

import numpy as np
import jax
import jax.numpy as jnp
from jax.experimental import pallas as pl
from jax.experimental.pallas import tpu as pltpu

MV = 16
RMS_EPS = 1e-6
ND_LANES = (0, 2, 3, 4, 8, 9, 10, 14)
TRI_LANES = (11, 12, 13)

_BLADES = [(), (0,), (1,), (2,), (3,), (0, 1), (0, 2), (0, 3), (1, 2), (1, 3),
           (2, 3), (0, 1, 2), (0, 1, 3), (0, 2, 3), (1, 2, 3), (0, 1, 2, 3)]
_B2I = {b: i for i, b in enumerate(_BLADES)}


def _perm_sign(seq):
    arr = list(seq)
    sgn = 1.0
    for a in range(1, len(arr)):
        b = a
        while b > 0 and arr[b - 1] > arr[b]:
            arr[b - 1], arr[b] = arr[b], arr[b - 1]
            sgn = -sgn
            b -= 1
    return sgn, arr


def _mul_blades(x, y):
    sgn, arr = _perm_sign(list(x) + list(y))
    out, i = [], 0
    while i < len(arr):
        if i + 1 < len(arr) and arr[i] == arr[i + 1]:
            if arr[i] == 0:
                return 0.0, ()
            i += 2
        else:
            out.append(arr[i])
            i += 1
    return sgn, tuple(out)


def _tables():
    gp = np.zeros((16, 16, 16), np.float32)
    wedge = np.zeros((16, 16, 16), np.float32)
    for i, a in enumerate(_BLADES):
        for j, b in enumerate(_BLADES):
            s, c = _mul_blades(a, b)
            if s:
                gp[i, j, _B2I[c]] = s
            if not (set(a) & set(b)):
                s2, arr = _perm_sign(list(a) + list(b))
                wedge[i, j, _B2I[tuple(arr)]] = s2
    dual = np.zeros((16, 16), np.float32)
    for i, bl in enumerate(_BLADES):
        comp = tuple(sorted(set((0, 1, 2, 3)) - set(bl)))
        s, _ = _perm_sign(list(bl) + list(comp))
        dual[_B2I[comp], i] = s
    join = np.einsum("mn,pqm,pi,qj->ijn", dual, wedge, dual, dual)
    return gp, join.astype(np.float32)


_GP_TBL, _JOIN_TBL = _tables()


def _term_list(tbl):
    out = [[] for _ in range(16)]
    for i, j, n in np.argwhere(tbl != 0.0):
        out[int(n)].append((int(i), int(j), float(tbl[i, j, n])))
    return out


_GP_TERMS = _term_list(_GP_TBL)
_JOIN_TERMS = _term_list(_JOIN_TBL)


def _qkv_kernel(x_ref, w_ref, b_ref, mask_ref, qs_ref, qc_ref, ks_ref, kc_ref,
                tq_ref, tk_ref, qf_ref, kf_ref, vf_ref, *, heads, cdim, inv_c):
    x = x_ref[0]
    ms = jnp.sum(x * x * mask_ref[...], axis=-1, keepdims=True) * inv_c
    xn = (x * jax.lax.rsqrt(ms + RMS_EPS)).astype(jnp.bfloat16)
    qkv = jnp.dot(xn, w_ref[...], preferred_element_type=jnp.float32)
    qkv = qkv + b_ref[...]
    tk = tk_ref[...]
    for h in range(heads):
        q = qkv[:, h * cdim:(h + 1) * cdim]
        k = qkv[:, (heads + h) * cdim:(heads + h + 1) * cdim]
        v = qkv[:, (2 * heads + h) * cdim:(2 * heads + h + 1) * cdim]
        qf = q * qs_ref[h][None, :] + qc_ref[h][None, :]
        qf = qf + jnp.dot((q * q).astype(jnp.bfloat16), tq_ref[h],
                          preferred_element_type=jnp.float32)
        kf = k * ks_ref[...] + kc_ref[...]
        kf = kf + jnp.dot((k * k).astype(jnp.bfloat16), tk,
                          preferred_element_type=jnp.float32)
        qf_ref[0, h] = qf.astype(jnp.bfloat16)
        kf_ref[0, h] = kf.astype(jnp.bfloat16)
        vf_ref[0, h] = v.astype(jnp.bfloat16)


def _attn_kernel(q_ref, k_ref, v_ref, o_ref, *, scale, tq, tk, dv):
    qi = pl.program_id(1)
    q = q_ref[0]

    def body(kj, carry):
        m, l, acc = carry
        kb = k_ref[0, pl.ds(kj * tk, tk), :]
        s = jax.lax.dot_general(q, kb, (((1,), (1,)), ((), ())),
                                preferred_element_type=jnp.float32) * scale
        row = qi * tq + jax.lax.broadcasted_iota(jnp.int32, (tq, tk), 0)
        col = kj * tk + jax.lax.broadcasted_iota(jnp.int32, (tq, tk), 1)
        s = jnp.where(col <= row, s, -1e30)
        m_new = jnp.maximum(m, jnp.max(s, axis=-1, keepdims=True))
        alpha = jnp.exp(m - m_new)
        p = jnp.exp(s - m_new)
        l_new = alpha * l + jnp.sum(p, axis=-1, keepdims=True)
        vb = v_ref[0, pl.ds(kj * tk, tk), :]
        acc_new = alpha * acc + jnp.dot(p.astype(jnp.bfloat16), vb,
                                        preferred_element_type=jnp.float32)
        return m_new, l_new, acc_new

    init = (jnp.full((tq, 1), -1e30, jnp.float32),
            jnp.zeros((tq, 1), jnp.float32),
            jnp.zeros((tq, dv), jnp.float32))
    m, l, acc = jax.lax.fori_loop(0, qi + 1, body, init)
    o_ref[0] = (acc / l).astype(jnp.bfloat16)


def _mid_kernel(attn_ref, x_ref, wo_ref, bo_ref, wb_ref, bb_ref, mask_ref,
                ps_ref, xa_ref, y_ref, *, heads, inv_c, y_dim, cdim):
    acc = jnp.dot(attn_ref[0, 0], wo_ref[0],
                  preferred_element_type=jnp.float32)
    for h in range(1, heads):
        acc = acc + jnp.dot(attn_ref[0, h], wo_ref[h],
                            preferred_element_type=jnp.float32)
    xa = acc + bo_ref[...] + x_ref[0]
    xa_ref[0] = xa
    ms = jnp.sum(xa * xa * mask_ref[...], axis=-1, keepdims=True) * inv_c
    xn = (xa * jax.lax.rsqrt(ms + RMS_EPS)).astype(jnp.bfloat16)
    y = jnp.dot(xn, wb_ref[...], preferred_element_type=jnp.float32)
    y = y + bb_ref[...]
    lane = jax.lax.broadcasted_iota(jnp.int32, (1, y_dim), 1)
    in_lj = jnp.logical_and(lane >= 2 * cdim, lane < 3 * cdim)
    y = y * jnp.where(in_lj, ps_ref[0, 0, 0], 1.0)
    y_ref[0] = y.astype(jnp.bfloat16)


def _bilinear_kernel(lg_ref, rg_ref, lj_ref, rj_ref, og_ref, oj_ref):
    for terms, a_ref, b_ref, o_ref in ((_GP_TERMS, lg_ref, rg_ref, og_ref),
                                       (_JOIN_TERMS, lj_ref, rj_ref, oj_ref)):
        a = a_ref[...].astype(jnp.float32)
        b = b_ref[...].astype(jnp.float32)
        for n in range(16):
            acc = None
            for (i, j, s) in terms[n]:
                t = a[i] * b[j]
                if s == -1.0:
                    t = -t
                elif s != 1.0:
                    t = t * s
                acc = t if acc is None else acc + t
            if acc is None:
                acc = jnp.zeros_like(a[0])
            o_ref[n] = acc.astype(jnp.bfloat16)


def _out_kernel(z_ref, w1_ref, b1_ref, w2_ref, b2_ref, res_ref, o_ref, *,
                cdim):
    t = jnp.dot(z_ref[...], w1_ref[...], preferred_element_type=jnp.float32)
    t = t + b1_ref[...]
    z2 = t[:, :cdim]
    gate = jax.nn.gelu(t[:, cdim:], approximate=True)
    gated = (z2 * gate).astype(jnp.bfloat16)
    out = jnp.dot(gated, w2_ref[...], preferred_element_type=jnp.float32)
    o_ref[...] = out + b2_ref[...] + res_ref[...]


def _feature_constants(w_ipa, w_daa, c_h):
    heads = w_ipa.shape[0]
    cdim = c_h * MV
    nd = np.zeros((MV,), np.float32)
    nd[list(ND_LANES)] = 1.0
    tri = np.zeros((MV,), np.float32)
    tri[list(TRI_LANES)] = 1.0
    lane1 = np.zeros((MV,), np.float32)
    lane1[1] = 1.0
    lane5 = np.zeros((MV,), np.float32)
    lane5[5] = 1.0

    ndj = jnp.asarray(nd)
    trij = jnp.asarray(tri)
    qscale = (w_ipa[:, :, None] * ndj + 2.0 * w_daa[:, :, None] * trij)
    qscale = qscale.reshape(heads, cdim)
    qconst = (-w_daa[:, :, None] * jnp.asarray(lane5)).reshape(heads, cdim)
    kscale = np.tile(nd + tri, c_h).reshape(1, cdim)
    kconst = np.tile(lane1, c_h).reshape(1, cdim)

    pat1 = np.zeros((cdim, cdim), np.float32)
    pat5 = np.zeros((cdim, cdim), np.float32)
    for c in range(c_h):
        for t in TRI_LANES:
            pat1[c * MV + t, c * MV + 1] = 1.0
            pat5[c * MV + t, c * MV + 5] = 1.0
    col_w = (-w_daa[:, :, None] * jnp.ones((1, 1, MV))).reshape(heads, 1, cdim)
    tq_mat = (jnp.asarray(pat1)[None] * col_w).astype(jnp.bfloat16)
    tk_mat = jnp.asarray(pat5).astype(jnp.bfloat16)
    return (qscale, qconst, jnp.asarray(kscale), jnp.asarray(kconst),
            tq_mat, tk_mat)


def kernel(x, ref_input, qkv_w, qkv_b, attn_out_w, attn_out_b, bil_w, bil_b,
           bil_out_w, bil_out_b, mlp_out_w, mlp_out_b, w_ipa, w_daa,
           norm_mask):
    b, t, c_h, mv = x.shape
    assert mv == MV
    heads = w_ipa.shape[0]
    cdim = c_h * MV
    c_inter = bil_w.shape[1] // (4 * MV)
    n = b * t
    inv_c = 1.0 / c_h
    scale = 1.0 / np.sqrt(c_h * 13)

    x3 = x.reshape(b, t, cdim)
    tt = min(256, t)
    nt = t // tt

    qs, qc, ks, kc, tq_mat, tk_mat = _feature_constants(w_ipa, w_daa, c_h)

    bf = jnp.bfloat16
    qf, kf, vf = pl.pallas_call(
        functools.partial(_qkv_kernel, heads=heads, cdim=cdim, inv_c=inv_c),
        out_shape=(jax.ShapeDtypeStruct((b, heads, t, cdim), bf),) * 3,
        grid=(b, nt),
        in_specs=[
            pl.BlockSpec((1, tt, cdim), lambda i, j: (i, j, 0)),
            pl.BlockSpec(qkv_w.shape, lambda i, j: (0, 0)),
            pl.BlockSpec(qkv_b.shape, lambda i, j: (0, 0)),
            pl.BlockSpec(norm_mask.shape, lambda i, j: (0, 0)),
            pl.BlockSpec((heads, cdim), lambda i, j: (0, 0)),
            pl.BlockSpec((heads, cdim), lambda i, j: (0, 0)),
            pl.BlockSpec((1, cdim), lambda i, j: (0, 0)),
            pl.BlockSpec((1, cdim), lambda i, j: (0, 0)),
            pl.BlockSpec((heads, cdim, cdim), lambda i, j: (0, 0, 0)),
            pl.BlockSpec((cdim, cdim), lambda i, j: (0, 0)),
        ],
        out_specs=(pl.BlockSpec((1, heads, tt, cdim),
                                lambda i, j: (i, 0, j, 0)),) * 3,
        compiler_params=pltpu.CompilerParams(
            dimension_semantics=("parallel", "parallel"),
            vmem_limit_bytes=64 * 1024 * 1024),
    )(x3, qkv_w.astype(bf), qkv_b, norm_mask, qs, qc, ks, kc, tq_mat, tk_mat)

    bh = b * heads
    tqb = min(128, t)
    nq = t // tqb
    qf = qf.reshape(bh, t, cdim)
    kf = kf.reshape(bh, t, cdim)
    vf = vf.reshape(bh, t, cdim)
    attn = pl.pallas_call(
        functools.partial(_attn_kernel, scale=scale, tq=tqb, tk=tqb, dv=cdim),
        out_shape=jax.ShapeDtypeStruct((bh, t, cdim), bf),
        grid=(bh, nq),
        in_specs=[
            pl.BlockSpec((1, tqb, cdim), lambda i, qi: (i, qi, 0)),
            pl.BlockSpec((1, t, cdim), lambda i, qi: (i, 0, 0)),
            pl.BlockSpec((1, t, cdim), lambda i, qi: (i, 0, 0)),
        ],
        out_specs=pl.BlockSpec((1, tqb, cdim), lambda i, qi: (i, qi, 0)),
        compiler_params=pltpu.CompilerParams(
            dimension_semantics=("parallel", "arbitrary"),
            vmem_limit_bytes=64 * 1024 * 1024),
    )(qf, kf, vf)
    attn = attn.reshape(b, heads, t, cdim)

    y_dim = 4 * c_inter * MV
    ref_ps = jnp.broadcast_to(ref_input[:, 0, 0, 15][:, None, None],
                              (b, 1, 128)).astype(jnp.float32)
    xa, y = pl.pallas_call(
        functools.partial(_mid_kernel, heads=heads, inv_c=inv_c,
                          y_dim=y_dim, cdim=cdim),
        out_shape=(jax.ShapeDtypeStruct((b, t, cdim), jnp.float32),
                   jax.ShapeDtypeStruct((b, t, y_dim), bf)),
        grid=(b, nt),
        in_specs=[
            pl.BlockSpec((1, heads, tt, cdim), lambda i, j: (i, 0, j, 0)),
            pl.BlockSpec((1, tt, cdim), lambda i, j: (i, j, 0)),
            pl.BlockSpec((heads, cdim, cdim), lambda i, j: (0, 0, 0)),
            pl.BlockSpec(attn_out_b.shape, lambda i, j: (0, 0)),
            pl.BlockSpec(bil_w.shape, lambda i, j: (0, 0)),
            pl.BlockSpec(bil_b.shape, lambda i, j: (0, 0)),
            pl.BlockSpec(norm_mask.shape, lambda i, j: (0, 0)),
            pl.BlockSpec((1, 1, 128), lambda i, j: (i, 0, 0)),
        ],
        out_specs=(pl.BlockSpec((1, tt, cdim), lambda i, j: (i, j, 0)),
                   pl.BlockSpec((1, tt, y_dim), lambda i, j: (i, j, 0))),
        compiler_params=pltpu.CompilerParams(
            dimension_semantics=("parallel", "parallel"),
            vmem_limit_bytes=64 * 1024 * 1024),
    )(attn, x3, attn_out_w.reshape(heads, cdim, cdim).astype(bf), attn_out_b,
      bil_w.astype(bf), bil_b, norm_mask, ref_ps)

    m = n * c_inter
    r = m // 128
    tr = min(256, r)
    y4 = y.reshape(n, 4, c_inter, MV)

    def to_bm(a):
        return jnp.transpose(a, (2, 0, 1)).reshape(MV, r, 128)

    lg, rg, lj, rj = (to_bm(y4[:, i]) for i in range(4))
    bspec = pl.BlockSpec((MV, tr, 128), lambda i: (0, i, 0))
    og, oj = pl.pallas_call(
        _bilinear_kernel,
        out_shape=(jax.ShapeDtypeStruct((MV, r, 128), bf),) * 2,
        grid=(r // tr,),
        in_specs=[bspec] * 4,
        out_specs=(bspec, bspec),
        compiler_params=pltpu.CompilerParams(
            dimension_semantics=("parallel",),
            vmem_limit_bytes=64 * 1024 * 1024),
    )(lg, rg, lj, rj)

    def from_bm(a):
        return jnp.transpose(a.reshape(MV, n, c_inter), (1, 2, 0))

    z = jnp.concatenate([from_bm(og), from_bm(oj)], axis=1)
    z = z.reshape(n, 2 * c_inter * MV)

    w1 = jnp.concatenate(
        [bil_out_w, jnp.repeat(bil_out_w[:, ::MV], MV, axis=1)], axis=1)
    b1 = jnp.concatenate(
        [bil_out_b, jnp.repeat(bil_out_b[:, ::MV], MV, axis=1)], axis=1)
    to = min(512, n)
    out = pl.pallas_call(
        functools.partial(_out_kernel, cdim=cdim),
        out_shape=jax.ShapeDtypeStruct((n, cdim), jnp.float32),
        grid=(n // to,),
        in_specs=[
            pl.BlockSpec((to, 2 * c_inter * MV), lambda i: (i, 0)),
            pl.BlockSpec((2 * c_inter * MV, 2 * cdim), lambda i: (0, 0)),
            pl.BlockSpec((1, 2 * cdim), lambda i: (0, 0)),
            pl.BlockSpec((cdim, cdim), lambda i: (0, 0)),
            pl.BlockSpec(mlp_out_b.shape, lambda i: (0, 0)),
            pl.BlockSpec((to, cdim), lambda i: (i, 0)),
        ],
        out_specs=pl.BlockSpec((to, cdim), lambda i: (i, 0)),
        compiler_params=pltpu.CompilerParams(
            dimension_semantics=("parallel",),
            vmem_limit_bytes=64 * 1024 * 1024),
    )(z, w1.astype(bf), b1, mlp_out_w.astype(bf), mlp_out_b,
      xa.reshape(n, cdim))

    return out.reshape(b, t, c_h, MV)


# speedup vs baseline: 4.3138x; 1.5734x over previous
import functools

import numpy as np
import jax
import jax.numpy as jnp
from jax.experimental import pallas as pl
from jax.experimental.pallas import tpu as pltpu

MV = 16
RMS_EPS = 1e-6
ND_LANES = (0, 2, 3, 4, 8, 9, 10, 14)
TRI_LANES = (11, 12, 13)

_BLADES = [(), (0,), (1,), (2,), (3,), (0, 1), (0, 2), (0, 3), (1, 2), (1, 3),
           (2, 3), (0, 1, 2), (0, 1, 3), (0, 2, 3), (1, 2, 3), (0, 1, 2, 3)]
_B2I = {b: i for i, b in enumerate(_BLADES)}


def _perm_sign(seq):
    arr = list(seq)
    sgn = 1.0
    for a in range(1, len(arr)):
        b = a
        while b > 0 and arr[b - 1] > arr[b]:
            arr[b - 1], arr[b] = arr[b], arr[b - 1]
            sgn = -sgn
            b -= 1
    return sgn, arr


def _mul_blades(x, y):
    sgn, arr = _perm_sign(list(x) + list(y))
    out, i = [], 0
    while i < len(arr):
        if i + 1 < len(arr) and arr[i] == arr[i + 1]:
            if arr[i] == 0:
                return 0.0, ()
            i += 2
        else:
            out.append(arr[i])
            i += 1
    return sgn, tuple(out)


def _tables():
    gp = np.zeros((16, 16, 16), np.float32)
    wedge = np.zeros((16, 16, 16), np.float32)
    for i, a in enumerate(_BLADES):
        for j, b in enumerate(_BLADES):
            s, c = _mul_blades(a, b)
            if s:
                gp[i, j, _B2I[c]] = s
            if not (set(a) & set(b)):
                s2, arr = _perm_sign(list(a) + list(b))
                wedge[i, j, _B2I[tuple(arr)]] = s2
    dual = np.zeros((16, 16), np.float32)
    for i, bl in enumerate(_BLADES):
        comp = tuple(sorted(set((0, 1, 2, 3)) - set(bl)))
        s, _ = _perm_sign(list(bl) + list(comp))
        dual[_B2I[comp], i] = s
    join = np.einsum("mn,pqm,pi,qj->ijn", dual, wedge, dual, dual)
    return gp, join.astype(np.float32)


_GP_TBL, _JOIN_TBL = _tables()


def _term_list(tbl):
    out = [[] for _ in range(16)]
    for i, j, n in np.argwhere(tbl != 0.0):
        out[int(n)].append((int(i), int(j), float(tbl[i, j, n])))
    return out


_GP_TERMS = _term_list(_GP_TBL)
_JOIN_TERMS = _term_list(_JOIN_TBL)


def _qkv_kernel(x_ref, w_ref, b_ref, mask_ref, qs_ref, qc_ref, ks_ref, kc_ref,
                tq_ref, tk_ref, qf_ref, kf_ref, vf_ref, *, heads, cdim, inv_c):
    x = x_ref[0]
    ms = jnp.sum(x * x * mask_ref[...], axis=-1, keepdims=True) * inv_c
    xn = (x * jax.lax.rsqrt(ms + RMS_EPS)).astype(jnp.bfloat16)
    qkv = jnp.dot(xn, w_ref[...], preferred_element_type=jnp.float32)
    qkv = qkv + b_ref[...]
    tk = tk_ref[...]
    for h in range(heads):
        q = qkv[:, h * cdim:(h + 1) * cdim]
        k = qkv[:, (heads + h) * cdim:(heads + h + 1) * cdim]
        v = qkv[:, (2 * heads + h) * cdim:(2 * heads + h + 1) * cdim]
        qf = q * qs_ref[h][None, :] + qc_ref[h][None, :]
        qf = qf + jnp.dot((q * q).astype(jnp.bfloat16), tq_ref[h],
                          preferred_element_type=jnp.float32)
        kf = k * ks_ref[...] + kc_ref[...]
        kf = kf + jnp.dot((k * k).astype(jnp.bfloat16), tk,
                          preferred_element_type=jnp.float32)
        qf_ref[0, h] = qf.astype(jnp.bfloat16)
        kf_ref[0, h] = kf.astype(jnp.bfloat16)
        vf_ref[0, h] = v.astype(jnp.bfloat16)


def _attn_kernel(q_ref, k_ref, v_ref, o_ref, *, scale, tq, tk, dv):
    qi = pl.program_id(1)
    q = q_ref[0]

    def body(kj, carry):
        m, l, acc = carry
        kb = k_ref[0, pl.ds(kj * tk, tk), :]
        s = jax.lax.dot_general(q, kb, (((1,), (1,)), ((), ())),
                                preferred_element_type=jnp.float32) * scale
        row = qi * tq + jax.lax.broadcasted_iota(jnp.int32, (tq, tk), 0)
        col = kj * tk + jax.lax.broadcasted_iota(jnp.int32, (tq, tk), 1)
        s = jnp.where(col <= row, s, -1e30)
        m_new = jnp.maximum(m, jnp.max(s, axis=-1, keepdims=True))
        alpha = jnp.exp(m - m_new)
        p = jnp.exp(s - m_new)
        l_new = alpha * l + jnp.sum(p, axis=-1, keepdims=True)
        vb = v_ref[0, pl.ds(kj * tk, tk), :]
        acc_new = alpha * acc + jnp.dot(p.astype(jnp.bfloat16), vb,
                                        preferred_element_type=jnp.float32)
        return m_new, l_new, acc_new

    init = (jnp.full((tq, 1), -1e30, jnp.float32),
            jnp.zeros((tq, 1), jnp.float32),
            jnp.zeros((tq, dv), jnp.float32))
    m, l, acc = jax.lax.fori_loop(0, qi + 1, body, init)
    o_ref[0] = (acc / l).astype(jnp.bfloat16)


def _tail_kernel(attn_ref, x_ref, wo_ref, bo_ref, wb_ref, bb_ref, mask_ref,
                 ps_ref, w1_ref, b1_ref, w2_ref, b2_ref, o_ref, *,
                 heads, inv_c, cdim, rows):
    acc = jnp.dot(attn_ref[0, 0], wo_ref[0],
                  preferred_element_type=jnp.float32)
    for h in range(1, heads):
        acc = acc + jnp.dot(attn_ref[0, h], wo_ref[h],
                            preferred_element_type=jnp.float32)
    xa = acc + bo_ref[...] + x_ref[0]
    ms = jnp.sum(xa * xa * mask_ref[...], axis=-1, keepdims=True) * inv_c
    xn = (xa * jax.lax.rsqrt(ms + RMS_EPS)).astype(jnp.bfloat16)
    y = jnp.dot(xn, wb_ref[...], preferred_element_type=jnp.float32)
    y = y + bb_ref[...]

    odim = wb_ref.shape[1] // 4
    c_i = odim // MV

    def to_bm(k):
        t = jnp.transpose(y[:, k * odim:(k + 1) * odim].astype(jnp.bfloat16))
        return t.reshape(c_i, MV, rows).astype(jnp.float32)

    lg, rg, rj = to_bm(0), to_bm(1), to_bm(3)
    lj = to_bm(2) * ps_ref[0, 0, 0]

    halves = []
    for terms, a, bb in ((_GP_TERMS, lg, rg), (_JOIN_TERMS, lj, rj)):
        outs = []
        for n in range(16):
            nacc = None
            for (i, j, s) in terms[n]:
                t = a[:, i, :] * bb[:, j, :]
                if s == -1.0:
                    t = -t
                elif s != 1.0:
                    t = t * s
                nacc = t if nacc is None else nacc + t
            outs.append(nacc if nacc is not None
                        else jnp.zeros_like(a[:, 0, :]))
        half = jnp.stack(outs, axis=1)
        half = jnp.transpose(half.reshape(odim, rows).astype(jnp.bfloat16))
        halves.append(half)
    z = jnp.concatenate(halves, axis=1)

    t2 = jnp.dot(z, w1_ref[...], preferred_element_type=jnp.float32)
    t2 = t2 + b1_ref[...]
    z2 = t2[:, :cdim]
    gate = jax.nn.gelu(t2[:, cdim:], approximate=True)
    gated = (z2 * gate).astype(jnp.bfloat16)
    out = jnp.dot(gated, w2_ref[...], preferred_element_type=jnp.float32)
    o_ref[0] = out + b2_ref[...] + xa


def _feature_constants(w_ipa, w_daa, c_h):
    heads = w_ipa.shape[0]
    cdim = c_h * MV
    nd = np.zeros((MV,), np.float32)
    nd[list(ND_LANES)] = 1.0
    tri = np.zeros((MV,), np.float32)
    tri[list(TRI_LANES)] = 1.0
    lane1 = np.zeros((MV,), np.float32)
    lane1[1] = 1.0
    lane5 = np.zeros((MV,), np.float32)
    lane5[5] = 1.0

    ndj = jnp.asarray(nd)
    trij = jnp.asarray(tri)
    qscale = (w_ipa[:, :, None] * ndj + 2.0 * w_daa[:, :, None] * trij)
    qscale = qscale.reshape(heads, cdim)
    qconst = (-w_daa[:, :, None] * jnp.asarray(lane5)).reshape(heads, cdim)
    kscale = np.tile(nd + tri, c_h).reshape(1, cdim)
    kconst = np.tile(lane1, c_h).reshape(1, cdim)

    pat1 = np.zeros((cdim, cdim), np.float32)
    pat5 = np.zeros((cdim, cdim), np.float32)
    for c in range(c_h):
        for t in TRI_LANES:
            pat1[c * MV + t, c * MV + 1] = 1.0
            pat5[c * MV + t, c * MV + 5] = 1.0
    col_w = (-w_daa[:, :, None] * jnp.ones((1, 1, MV))).reshape(heads, 1, cdim)
    tq_mat = (jnp.asarray(pat1)[None] * col_w).astype(jnp.bfloat16)
    tk_mat = jnp.asarray(pat5).astype(jnp.bfloat16)
    return (qscale, qconst, jnp.asarray(kscale), jnp.asarray(kconst),
            tq_mat, tk_mat)


def kernel(x, ref_input, qkv_w, qkv_b, attn_out_w, attn_out_b, bil_w, bil_b,
           bil_out_w, bil_out_b, mlp_out_w, mlp_out_b, w_ipa, w_daa,
           norm_mask):
    b, t, c_h, mv = x.shape
    assert mv == MV
    heads = w_ipa.shape[0]
    cdim = c_h * MV
    c_inter = bil_w.shape[1] // (4 * MV)
    n = b * t
    inv_c = 1.0 / c_h
    scale = 1.0 / np.sqrt(c_h * 13)

    x3 = x.reshape(b, t, cdim)
    tt = min(256, t)
    nt = t // tt

    qs, qc, ks, kc, tq_mat, tk_mat = _feature_constants(w_ipa, w_daa, c_h)

    bf = jnp.bfloat16
    qf, kf, vf = pl.pallas_call(
        functools.partial(_qkv_kernel, heads=heads, cdim=cdim, inv_c=inv_c),
        out_shape=(jax.ShapeDtypeStruct((b, heads, t, cdim), bf),) * 3,
        grid=(b, nt),
        in_specs=[
            pl.BlockSpec((1, tt, cdim), lambda i, j: (i, j, 0)),
            pl.BlockSpec(qkv_w.shape, lambda i, j: (0, 0)),
            pl.BlockSpec(qkv_b.shape, lambda i, j: (0, 0)),
            pl.BlockSpec(norm_mask.shape, lambda i, j: (0, 0)),
            pl.BlockSpec((heads, cdim), lambda i, j: (0, 0)),
            pl.BlockSpec((heads, cdim), lambda i, j: (0, 0)),
            pl.BlockSpec((1, cdim), lambda i, j: (0, 0)),
            pl.BlockSpec((1, cdim), lambda i, j: (0, 0)),
            pl.BlockSpec((heads, cdim, cdim), lambda i, j: (0, 0, 0)),
            pl.BlockSpec((cdim, cdim), lambda i, j: (0, 0)),
        ],
        out_specs=(pl.BlockSpec((1, heads, tt, cdim),
                                lambda i, j: (i, 0, j, 0)),) * 3,
        compiler_params=pltpu.CompilerParams(
            dimension_semantics=("parallel", "parallel"),
            vmem_limit_bytes=64 * 1024 * 1024),
    )(x3, qkv_w.astype(bf), qkv_b, norm_mask, qs, qc, ks, kc, tq_mat, tk_mat)

    bh = b * heads
    tqb = min(128, t)
    nq = t // tqb
    qf = qf.reshape(bh, t, cdim)
    kf = kf.reshape(bh, t, cdim)
    vf = vf.reshape(bh, t, cdim)
    attn = pl.pallas_call(
        functools.partial(_attn_kernel, scale=scale, tq=tqb, tk=tqb, dv=cdim),
        out_shape=jax.ShapeDtypeStruct((bh, t, cdim), bf),
        grid=(bh, nq),
        in_specs=[
            pl.BlockSpec((1, tqb, cdim), lambda i, qi: (i, qi, 0)),
            pl.BlockSpec((1, t, cdim), lambda i, qi: (i, 0, 0)),
            pl.BlockSpec((1, t, cdim), lambda i, qi: (i, 0, 0)),
        ],
        out_specs=pl.BlockSpec((1, tqb, cdim), lambda i, qi: (i, qi, 0)),
        compiler_params=pltpu.CompilerParams(
            dimension_semantics=("parallel", "arbitrary"),
            vmem_limit_bytes=64 * 1024 * 1024),
    )(qf, kf, vf)
    attn = attn.reshape(b, heads, t, cdim)

    ref_ps = jnp.broadcast_to(ref_input[:, 0, 0, 15][:, None, None],
                              (b, 1, 128)).astype(jnp.float32)
    w1 = jnp.concatenate(
        [bil_out_w, jnp.repeat(bil_out_w[:, ::MV], MV, axis=1)], axis=1)
    b1 = jnp.concatenate(
        [bil_out_b, jnp.repeat(bil_out_b[:, ::MV], MV, axis=1)], axis=1)
    zdim = 2 * c_inter * MV
    out = pl.pallas_call(
        functools.partial(_tail_kernel, heads=heads, inv_c=inv_c,
                          cdim=cdim, rows=tt),
        out_shape=jax.ShapeDtypeStruct((b, t, cdim), jnp.float32),
        grid=(b, nt),
        in_specs=[
            pl.BlockSpec((1, heads, tt, cdim), lambda i, j: (i, 0, j, 0)),
            pl.BlockSpec((1, tt, cdim), lambda i, j: (i, j, 0)),
            pl.BlockSpec((heads, cdim, cdim), lambda i, j: (0, 0, 0)),
            pl.BlockSpec(attn_out_b.shape, lambda i, j: (0, 0)),
            pl.BlockSpec(bil_w.shape, lambda i, j: (0, 0)),
            pl.BlockSpec(bil_b.shape, lambda i, j: (0, 0)),
            pl.BlockSpec(norm_mask.shape, lambda i, j: (0, 0)),
            pl.BlockSpec((1, 1, 128), lambda i, j: (i, 0, 0)),
            pl.BlockSpec((zdim, 2 * cdim), lambda i, j: (0, 0)),
            pl.BlockSpec((1, 2 * cdim), lambda i, j: (0, 0)),
            pl.BlockSpec((cdim, cdim), lambda i, j: (0, 0)),
            pl.BlockSpec(mlp_out_b.shape, lambda i, j: (0, 0)),
        ],
        out_specs=pl.BlockSpec((1, tt, cdim), lambda i, j: (i, j, 0)),
        compiler_params=pltpu.CompilerParams(
            dimension_semantics=("parallel", "parallel"),
            vmem_limit_bytes=64 * 1024 * 1024),
    )(attn, x3, attn_out_w.reshape(heads, cdim, cdim).astype(bf), attn_out_b,
      bil_w.astype(bf), bil_b, norm_mask, ref_ps, w1.astype(bf), b1,
      mlp_out_w.astype(bf), mlp_out_b)

    return out.reshape(b, t, c_h, MV)


# speedup vs baseline: 6.6363x; 1.5384x over previous
import functools

import numpy as np
import jax
import jax.numpy as jnp
from jax.experimental import pallas as pl
from jax.experimental.pallas import tpu as pltpu

MV = 16
RMS_EPS = 1e-6
ND_LANES = (0, 2, 3, 4, 8, 9, 10, 14)
TRI_LANES = (11, 12, 13)

_BLADES = [(), (0,), (1,), (2,), (3,), (0, 1), (0, 2), (0, 3), (1, 2), (1, 3),
           (2, 3), (0, 1, 2), (0, 1, 3), (0, 2, 3), (1, 2, 3), (0, 1, 2, 3)]
_B2I = {b: i for i, b in enumerate(_BLADES)}


def _perm_sign(seq):
    arr = list(seq)
    sgn = 1.0
    for a in range(1, len(arr)):
        b = a
        while b > 0 and arr[b - 1] > arr[b]:
            arr[b - 1], arr[b] = arr[b], arr[b - 1]
            sgn = -sgn
            b -= 1
    return sgn, arr


def _mul_blades(x, y):
    sgn, arr = _perm_sign(list(x) + list(y))
    out, i = [], 0
    while i < len(arr):
        if i + 1 < len(arr) and arr[i] == arr[i + 1]:
            if arr[i] == 0:
                return 0.0, ()
            i += 2
        else:
            out.append(arr[i])
            i += 1
    return sgn, tuple(out)


def _tables():
    gp = np.zeros((16, 16, 16), np.float32)
    wedge = np.zeros((16, 16, 16), np.float32)
    for i, a in enumerate(_BLADES):
        for j, b in enumerate(_BLADES):
            s, c = _mul_blades(a, b)
            if s:
                gp[i, j, _B2I[c]] = s
            if not (set(a) & set(b)):
                s2, arr = _perm_sign(list(a) + list(b))
                wedge[i, j, _B2I[tuple(arr)]] = s2
    dual = np.zeros((16, 16), np.float32)
    for i, bl in enumerate(_BLADES):
        comp = tuple(sorted(set((0, 1, 2, 3)) - set(bl)))
        s, _ = _perm_sign(list(bl) + list(comp))
        dual[_B2I[comp], i] = s
    join = np.einsum("mn,pqm,pi,qj->ijn", dual, wedge, dual, dual)
    return gp, join.astype(np.float32)


_GP_TBL, _JOIN_TBL = _tables()


def _term_list(tbl):
    out = [[] for _ in range(16)]
    for i, j, n in np.argwhere(tbl != 0.0):
        out[int(n)].append((int(i), int(j), float(tbl[i, j, n])))
    return out


_GP_TERMS = _term_list(_GP_TBL)
_JOIN_TERMS = _term_list(_JOIN_TBL)


def _qkv_kernel(x_ref, w_ref, b_ref, mask_ref, qs_ref, qc_ref, ks_ref, kc_ref,
                tq_ref, tk_ref, qf_ref, kf_ref, vf_ref, *, heads, cdim, inv_c):
    x = x_ref[0]
    ms = jnp.sum(x * x * mask_ref[...], axis=-1, keepdims=True) * inv_c
    xn = (x * jax.lax.rsqrt(ms + RMS_EPS)).astype(jnp.bfloat16)
    qkv = jnp.dot(xn, w_ref[...], preferred_element_type=jnp.float32)
    qkv = qkv + b_ref[...]
    tk = tk_ref[...]
    for h in range(heads):
        q = qkv[:, h * cdim:(h + 1) * cdim]
        k = qkv[:, (heads + h) * cdim:(heads + h + 1) * cdim]
        v = qkv[:, (2 * heads + h) * cdim:(2 * heads + h + 1) * cdim]
        qf = q * qs_ref[h][None, :] + qc_ref[h][None, :]
        qf = qf + jnp.dot((q * q).astype(jnp.bfloat16), tq_ref[h],
                          preferred_element_type=jnp.float32)
        kf = k * ks_ref[...] + kc_ref[...]
        kf = kf + jnp.dot((k * k).astype(jnp.bfloat16), tk,
                          preferred_element_type=jnp.float32)
        qf_ref[0, h] = qf.astype(jnp.bfloat16)
        kf_ref[0, h] = kf.astype(jnp.bfloat16)
        vf_ref[0, h] = v.astype(jnp.bfloat16)


def _attn_kernel(q_ref, k_ref, v_ref, o_ref, *, scale, seq):
    q = q_ref[0]
    s = jax.lax.dot_general(q, k_ref[0], (((1,), (1,)), ((), ())),
                            preferred_element_type=jnp.float32) * scale
    row = jax.lax.broadcasted_iota(jnp.int32, (seq, seq), 0)
    col = jax.lax.broadcasted_iota(jnp.int32, (seq, seq), 1)
    s = jnp.where(col <= row, s, -1e30)
    m = jnp.max(s, axis=-1, keepdims=True)
    p = jnp.exp(s - m)
    l = jnp.sum(p, axis=-1, keepdims=True)
    acc = jnp.dot(p.astype(jnp.bfloat16), v_ref[0],
                  preferred_element_type=jnp.float32)
    o_ref[0] = (acc / l).astype(jnp.bfloat16)


def _tail_kernel(attn_ref, x_ref, wo_ref, bo_ref, wb_ref, bb_ref, mask_ref,
                 ps_ref, w1_ref, b1_ref, w2_ref, b2_ref, o_ref, *,
                 heads, inv_c, cdim, rows):
    acc = jnp.dot(attn_ref[0, 0], wo_ref[0],
                  preferred_element_type=jnp.float32)
    for h in range(1, heads):
        acc = acc + jnp.dot(attn_ref[0, h], wo_ref[h],
                            preferred_element_type=jnp.float32)
    xa = acc + bo_ref[...] + x_ref[0]
    ms = jnp.sum(xa * xa * mask_ref[...], axis=-1, keepdims=True) * inv_c
    xn = (xa * jax.lax.rsqrt(ms + RMS_EPS)).astype(jnp.bfloat16)
    y = jnp.dot(xn, wb_ref[...], preferred_element_type=jnp.float32)
    y = y + bb_ref[...]

    odim = wb_ref.shape[1] // 4
    c_i = odim // MV

    def to_bm(k):
        t = jnp.transpose(y[:, k * odim:(k + 1) * odim].astype(jnp.bfloat16))
        return t.reshape(c_i, MV, rows).astype(jnp.float32)

    lg, rg, rj = to_bm(0), to_bm(1), to_bm(3)
    lj = to_bm(2) * ps_ref[0, 0, 0]

    halves = []
    for terms, a, bb in ((_GP_TERMS, lg, rg), (_JOIN_TERMS, lj, rj)):
        outs = []
        for n in range(16):
            nacc = None
            for (i, j, s) in terms[n]:
                t = a[:, i, :] * bb[:, j, :]
                if s == -1.0:
                    t = -t
                elif s != 1.0:
                    t = t * s
                nacc = t if nacc is None else nacc + t
            outs.append(nacc if nacc is not None
                        else jnp.zeros_like(a[:, 0, :]))
        half = jnp.stack(outs, axis=1)
        half = jnp.transpose(half.reshape(odim, rows).astype(jnp.bfloat16))
        halves.append(half)
    z = jnp.concatenate(halves, axis=1)

    t2 = jnp.dot(z, w1_ref[...], preferred_element_type=jnp.float32)
    t2 = t2 + b1_ref[...]
    z2 = t2[:, :cdim]
    gate = jax.nn.gelu(t2[:, cdim:], approximate=True)
    gated = (z2 * gate).astype(jnp.bfloat16)
    out = jnp.dot(gated, w2_ref[...], preferred_element_type=jnp.float32)
    o_ref[0] = out + b2_ref[...] + xa


def _feature_constants(w_ipa, w_daa, c_h):
    heads = w_ipa.shape[0]
    cdim = c_h * MV
    nd = np.zeros((MV,), np.float32)
    nd[list(ND_LANES)] = 1.0
    tri = np.zeros((MV,), np.float32)
    tri[list(TRI_LANES)] = 1.0
    lane1 = np.zeros((MV,), np.float32)
    lane1[1] = 1.0
    lane5 = np.zeros((MV,), np.float32)
    lane5[5] = 1.0

    ndj = jnp.asarray(nd)
    trij = jnp.asarray(tri)
    qscale = (w_ipa[:, :, None] * ndj + 2.0 * w_daa[:, :, None] * trij)
    qscale = qscale.reshape(heads, cdim)
    qconst = (-w_daa[:, :, None] * jnp.asarray(lane5)).reshape(heads, cdim)
    kscale = np.tile(nd + tri, c_h).reshape(1, cdim)
    kconst = np.tile(lane1, c_h).reshape(1, cdim)

    pat1 = np.zeros((cdim, cdim), np.float32)
    pat5 = np.zeros((cdim, cdim), np.float32)
    for c in range(c_h):
        for t in TRI_LANES:
            pat1[c * MV + t, c * MV + 1] = 1.0
            pat5[c * MV + t, c * MV + 5] = 1.0
    col_w = (-w_daa[:, :, None] * jnp.ones((1, 1, MV))).reshape(heads, 1, cdim)
    tq_mat = (jnp.asarray(pat1)[None] * col_w).astype(jnp.bfloat16)
    tk_mat = jnp.asarray(pat5).astype(jnp.bfloat16)
    return (qscale, qconst, jnp.asarray(kscale), jnp.asarray(kconst),
            tq_mat, tk_mat)


def kernel(x, ref_input, qkv_w, qkv_b, attn_out_w, attn_out_b, bil_w, bil_b,
           bil_out_w, bil_out_b, mlp_out_w, mlp_out_b, w_ipa, w_daa,
           norm_mask):
    b, t, c_h, mv = x.shape
    assert mv == MV
    heads = w_ipa.shape[0]
    cdim = c_h * MV
    c_inter = bil_w.shape[1] // (4 * MV)
    n = b * t
    inv_c = 1.0 / c_h
    scale = 1.0 / np.sqrt(c_h * 13)

    x3 = x.reshape(b, t, cdim)
    tt = min(256, t)
    nt = t // tt

    qs, qc, ks, kc, tq_mat, tk_mat = _feature_constants(w_ipa, w_daa, c_h)

    bf = jnp.bfloat16
    qf, kf, vf = pl.pallas_call(
        functools.partial(_qkv_kernel, heads=heads, cdim=cdim, inv_c=inv_c),
        out_shape=(jax.ShapeDtypeStruct((b, heads, t, cdim), bf),) * 3,
        grid=(b, nt),
        in_specs=[
            pl.BlockSpec((1, tt, cdim), lambda i, j: (i, j, 0)),
            pl.BlockSpec(qkv_w.shape, lambda i, j: (0, 0)),
            pl.BlockSpec(qkv_b.shape, lambda i, j: (0, 0)),
            pl.BlockSpec(norm_mask.shape, lambda i, j: (0, 0)),
            pl.BlockSpec((heads, cdim), lambda i, j: (0, 0)),
            pl.BlockSpec((heads, cdim), lambda i, j: (0, 0)),
            pl.BlockSpec((1, cdim), lambda i, j: (0, 0)),
            pl.BlockSpec((1, cdim), lambda i, j: (0, 0)),
            pl.BlockSpec((heads, cdim, cdim), lambda i, j: (0, 0, 0)),
            pl.BlockSpec((cdim, cdim), lambda i, j: (0, 0)),
        ],
        out_specs=(pl.BlockSpec((1, heads, tt, cdim),
                                lambda i, j: (i, 0, j, 0)),) * 3,
        compiler_params=pltpu.CompilerParams(
            dimension_semantics=("parallel", "parallel"),
            vmem_limit_bytes=64 * 1024 * 1024),
    )(x3, qkv_w.astype(bf), qkv_b, norm_mask, qs, qc, ks, kc, tq_mat, tk_mat)

    bh = b * heads
    qf = qf.reshape(bh, t, cdim)
    kf = kf.reshape(bh, t, cdim)
    vf = vf.reshape(bh, t, cdim)
    attn = pl.pallas_call(
        functools.partial(_attn_kernel, scale=scale, seq=t),
        out_shape=jax.ShapeDtypeStruct((bh, t, cdim), bf),
        grid=(bh,),
        in_specs=[
            pl.BlockSpec((1, t, cdim), lambda i: (i, 0, 0)),
            pl.BlockSpec((1, t, cdim), lambda i: (i, 0, 0)),
            pl.BlockSpec((1, t, cdim), lambda i: (i, 0, 0)),
        ],
        out_specs=pl.BlockSpec((1, t, cdim), lambda i: (i, 0, 0)),
        compiler_params=pltpu.CompilerParams(
            dimension_semantics=("parallel",),
            vmem_limit_bytes=64 * 1024 * 1024),
    )(qf, kf, vf)
    attn = attn.reshape(b, heads, t, cdim)

    ref_ps = jnp.broadcast_to(ref_input[:, 0, 0, 15][:, None, None],
                              (b, 1, 128)).astype(jnp.float32)
    w1 = jnp.concatenate(
        [bil_out_w, jnp.repeat(bil_out_w[:, ::MV], MV, axis=1)], axis=1)
    b1 = jnp.concatenate(
        [bil_out_b, jnp.repeat(bil_out_b[:, ::MV], MV, axis=1)], axis=1)
    zdim = 2 * c_inter * MV
    out = pl.pallas_call(
        functools.partial(_tail_kernel, heads=heads, inv_c=inv_c,
                          cdim=cdim, rows=tt),
        out_shape=jax.ShapeDtypeStruct((b, t, cdim), jnp.float32),
        grid=(b, nt),
        in_specs=[
            pl.BlockSpec((1, heads, tt, cdim), lambda i, j: (i, 0, j, 0)),
            pl.BlockSpec((1, tt, cdim), lambda i, j: (i, j, 0)),
            pl.BlockSpec((heads, cdim, cdim), lambda i, j: (0, 0, 0)),
            pl.BlockSpec(attn_out_b.shape, lambda i, j: (0, 0)),
            pl.BlockSpec(bil_w.shape, lambda i, j: (0, 0)),
            pl.BlockSpec(bil_b.shape, lambda i, j: (0, 0)),
            pl.BlockSpec(norm_mask.shape, lambda i, j: (0, 0)),
            pl.BlockSpec((1, 1, 128), lambda i, j: (i, 0, 0)),
            pl.BlockSpec((zdim, 2 * cdim), lambda i, j: (0, 0)),
            pl.BlockSpec((1, 2 * cdim), lambda i, j: (0, 0)),
            pl.BlockSpec((cdim, cdim), lambda i, j: (0, 0)),
            pl.BlockSpec(mlp_out_b.shape, lambda i, j: (0, 0)),
        ],
        out_specs=pl.BlockSpec((1, tt, cdim), lambda i, j: (i, j, 0)),
        compiler_params=pltpu.CompilerParams(
            dimension_semantics=("parallel", "parallel"),
            vmem_limit_bytes=64 * 1024 * 1024),
    )(attn, x3, attn_out_w.reshape(heads, cdim, cdim).astype(bf), attn_out_b,
      bil_w.astype(bf), bil_b, norm_mask, ref_ps, w1.astype(bf), b1,
      mlp_out_w.astype(bf), mlp_out_b)

    return out.reshape(b, t, c_h, MV)


# speedup vs baseline: 7.3643x; 1.1097x over previous
import functools

import numpy as np
import jax
import jax.numpy as jnp
from jax.experimental import pallas as pl
from jax.experimental.pallas import tpu as pltpu

MV = 16
RMS_EPS = 1e-6
ND_LANES = (0, 2, 3, 4, 8, 9, 10, 14)
TRI_LANES = (11, 12, 13)

_BLADES = [(), (0,), (1,), (2,), (3,), (0, 1), (0, 2), (0, 3), (1, 2), (1, 3),
           (2, 3), (0, 1, 2), (0, 1, 3), (0, 2, 3), (1, 2, 3), (0, 1, 2, 3)]
_B2I = {b: i for i, b in enumerate(_BLADES)}


def _perm_sign(seq):
    arr = list(seq)
    sgn = 1.0
    for a in range(1, len(arr)):
        b = a
        while b > 0 and arr[b - 1] > arr[b]:
            arr[b - 1], arr[b] = arr[b], arr[b - 1]
            sgn = -sgn
            b -= 1
    return sgn, arr


def _mul_blades(x, y):
    sgn, arr = _perm_sign(list(x) + list(y))
    out, i = [], 0
    while i < len(arr):
        if i + 1 < len(arr) and arr[i] == arr[i + 1]:
            if arr[i] == 0:
                return 0.0, ()
            i += 2
        else:
            out.append(arr[i])
            i += 1
    return sgn, tuple(out)


def _tables():
    gp = np.zeros((16, 16, 16), np.float32)
    wedge = np.zeros((16, 16, 16), np.float32)
    for i, a in enumerate(_BLADES):
        for j, b in enumerate(_BLADES):
            s, c = _mul_blades(a, b)
            if s:
                gp[i, j, _B2I[c]] = s
            if not (set(a) & set(b)):
                s2, arr = _perm_sign(list(a) + list(b))
                wedge[i, j, _B2I[tuple(arr)]] = s2
    dual = np.zeros((16, 16), np.float32)
    for i, bl in enumerate(_BLADES):
        comp = tuple(sorted(set((0, 1, 2, 3)) - set(bl)))
        s, _ = _perm_sign(list(bl) + list(comp))
        dual[_B2I[comp], i] = s
    join = np.einsum("mn,pqm,pi,qj->ijn", dual, wedge, dual, dual)
    return gp, join.astype(np.float32)


_GP_TBL, _JOIN_TBL = _tables()


def _term_list(tbl):
    out = [[] for _ in range(16)]
    for i, j, n in np.argwhere(tbl != 0.0):
        out[int(n)].append((int(i), int(j), float(tbl[i, j, n])))
    return out


_GP_TERMS = _term_list(_GP_TBL)
_JOIN_TERMS = _term_list(_JOIN_TBL)


def _block_kernel(x_ref, w_ref, b_ref, mask_ref, qs_ref, qc_ref, ks_ref,
                  kc_ref, inj_ref, m1_ref, m5_ref, wo_ref, bo_ref, wb_ref,
                  bb_ref, ps_ref, w1_ref, b1_ref, w2_ref, b2_ref, o_ref, *,
                  heads, cdim, inv_c, seq, scale):
    bf = jnp.bfloat16
    x = x_ref[0]
    ms = jnp.sum(x * x * mask_ref[...], axis=-1, keepdims=True) * inv_c
    xn = (x * jax.lax.rsqrt(ms + RMS_EPS)).astype(bf)
    qkv = jnp.dot(xn, w_ref[...], preferred_element_type=jnp.float32)
    qkv = qkv + b_ref[...]

    row = jax.lax.broadcasted_iota(jnp.int32, (seq, seq), 0)
    col = jax.lax.broadcasted_iota(jnp.int32, (seq, seq), 1)
    causal = col <= row

    acc = None
    for h in range(heads):
        q = qkv[:, h * cdim:(h + 1) * cdim]
        k = qkv[:, (heads + h) * cdim:(heads + h + 1) * cdim]
        v = qkv[:, (2 * heads + h) * cdim:(2 * heads + h + 1) * cdim]
        sq = jnp.concatenate([q * q, k * k], axis=1).astype(bf)
        inj = jnp.dot(sq, inj_ref[h], preferred_element_type=jnp.float32)
        qf = (q * qs_ref[h][None, :] + qc_ref[h][None, :]
              + inj * m1_ref[...]).astype(bf)
        kf = (k * ks_ref[...] + kc_ref[...] + inj * m5_ref[...]).astype(bf)
        s = jax.lax.dot_general(qf, kf, (((1,), (1,)), ((), ())),
                                preferred_element_type=jnp.float32) * scale
        s = jnp.where(causal, s, -1e30)
        m = jnp.max(s, axis=-1, keepdims=True)
        p = jnp.exp(s - m)
        l = jnp.sum(p, axis=-1, keepdims=True)
        attn_h = jnp.dot(p.astype(bf), v.astype(bf),
                         preferred_element_type=jnp.float32) / l
        part = jnp.dot(attn_h.astype(bf), wo_ref[h],
                       preferred_element_type=jnp.float32)
        acc = part if acc is None else acc + part

    xa = acc + bo_ref[...] + x
    ms2 = jnp.sum(xa * xa * mask_ref[...], axis=-1, keepdims=True) * inv_c
    xn2 = (xa * jax.lax.rsqrt(ms2 + RMS_EPS)).astype(bf)
    y = jnp.dot(xn2, wb_ref[...], preferred_element_type=jnp.float32)
    y = y + bb_ref[...]

    odim = wb_ref.shape[1] // 4
    c_i = odim // MV

    def to_bm(kk):
        tt = jnp.transpose(y[:, kk * odim:(kk + 1) * odim].astype(bf))
        return tt.reshape(c_i, MV, seq).astype(jnp.float32)

    lg, rg, rj = to_bm(0), to_bm(1), to_bm(3)
    lj = to_bm(2) * ps_ref[0, 0, 0]

    halves = []
    for terms, a, bb2 in ((_GP_TERMS, lg, rg), (_JOIN_TERMS, lj, rj)):
        outs = []
        for n in range(16):
            nacc = None
            for (i, j, sgn) in terms[n]:
                t = a[:, i, :] * bb2[:, j, :]
                if sgn == -1.0:
                    t = -t
                elif sgn != 1.0:
                    t = t * sgn
                nacc = t if nacc is None else nacc + t
            outs.append(nacc if nacc is not None
                        else jnp.zeros_like(a[:, 0, :]))
        half = jnp.stack(outs, axis=1)
        half = jnp.transpose(half.reshape(odim, seq).astype(bf))
        halves.append(half)
    z = jnp.concatenate(halves, axis=1)

    t2 = jnp.dot(z, w1_ref[...], preferred_element_type=jnp.float32)
    t2 = t2 + b1_ref[...]
    z2 = t2[:, :cdim]
    gate = jax.nn.gelu(t2[:, cdim:], approximate=True)
    gated = (z2 * gate).astype(bf)
    out = jnp.dot(gated, w2_ref[...], preferred_element_type=jnp.float32)
    o_ref[0] = out + b2_ref[...] + xa


def _feature_constants(w_ipa, w_daa, c_h):
    heads = w_ipa.shape[0]
    cdim = c_h * MV
    nd = np.zeros((MV,), np.float32)
    nd[list(ND_LANES)] = 1.0
    tri = np.zeros((MV,), np.float32)
    tri[list(TRI_LANES)] = 1.0
    lane1 = np.zeros((MV,), np.float32)
    lane1[1] = 1.0
    lane5 = np.zeros((MV,), np.float32)
    lane5[5] = 1.0

    ndj = jnp.asarray(nd)
    trij = jnp.asarray(tri)
    qscale = (w_ipa[:, :, None] * ndj + 2.0 * w_daa[:, :, None] * trij)
    qscale = qscale.reshape(heads, cdim)
    qconst = (-w_daa[:, :, None] * jnp.asarray(lane5)).reshape(heads, cdim)
    kscale = np.tile(nd + tri, c_h).reshape(1, cdim)
    kconst = np.tile(lane1, c_h).reshape(1, cdim)

    pat1 = np.zeros((cdim, cdim), np.float32)
    pat5 = np.zeros((cdim, cdim), np.float32)
    for c in range(c_h):
        for t in TRI_LANES:
            pat1[c * MV + t, c * MV + 1] = 1.0
            pat5[c * MV + t, c * MV + 5] = 1.0
    col_w = (-w_daa[:, :, None] * jnp.ones((1, 1, MV))).reshape(heads, 1, cdim)
    top = jnp.asarray(pat1)[None] * col_w
    bot = jnp.broadcast_to(jnp.asarray(pat5)[None], top.shape)
    inj = jnp.concatenate([top, bot], axis=1).astype(jnp.bfloat16)
    m1 = np.tile(lane1, c_h).reshape(1, cdim)
    m5 = np.tile(lane5, c_h).reshape(1, cdim)
    return (qscale, qconst, jnp.asarray(kscale), jnp.asarray(kconst),
            inj, jnp.asarray(m1), jnp.asarray(m5))


def kernel(x, ref_input, qkv_w, qkv_b, attn_out_w, attn_out_b, bil_w, bil_b,
           bil_out_w, bil_out_b, mlp_out_w, mlp_out_b, w_ipa, w_daa,
           norm_mask):
    b, t, c_h, mv = x.shape
    assert mv == MV
    heads = w_ipa.shape[0]
    cdim = c_h * MV
    c_inter = bil_w.shape[1] // (4 * MV)
    inv_c = 1.0 / c_h
    scale = 1.0 / np.sqrt(c_h * 13)

    x3 = x.reshape(b, t, cdim)
    qs, qc, ks, kc, inj, m1, m5 = _feature_constants(w_ipa, w_daa, c_h)
    ref_ps = jnp.broadcast_to(ref_input[:, 0, 0, 15][:, None, None],
                              (b, 1, 128)).astype(jnp.float32)
    w1 = jnp.concatenate(
        [bil_out_w, jnp.repeat(bil_out_w[:, ::MV], MV, axis=1)], axis=1)
    b1 = jnp.concatenate(
        [bil_out_b, jnp.repeat(bil_out_b[:, ::MV], MV, axis=1)], axis=1)
    zdim = 2 * c_inter * MV

    bf = jnp.bfloat16
    full = lambda a: pl.BlockSpec(a.shape, lambda i: tuple(0 for _ in a.shape))
    out = pl.pallas_call(
        functools.partial(_block_kernel, heads=heads, cdim=cdim, inv_c=inv_c,
                          seq=t, scale=scale),
        out_shape=jax.ShapeDtypeStruct((b, t, cdim), jnp.float32),
        grid=(b,),
        in_specs=[
            pl.BlockSpec((1, t, cdim), lambda i: (i, 0, 0)),
            pl.BlockSpec(qkv_w.shape, lambda i: (0, 0)),
            pl.BlockSpec(qkv_b.shape, lambda i: (0, 0)),
            pl.BlockSpec(norm_mask.shape, lambda i: (0, 0)),
            pl.BlockSpec((heads, cdim), lambda i: (0, 0)),
            pl.BlockSpec((heads, cdim), lambda i: (0, 0)),
            pl.BlockSpec((1, cdim), lambda i: (0, 0)),
            pl.BlockSpec((1, cdim), lambda i: (0, 0)),
            pl.BlockSpec((heads, 2 * cdim, cdim), lambda i: (0, 0, 0)),
            pl.BlockSpec((1, cdim), lambda i: (0, 0)),
            pl.BlockSpec((1, cdim), lambda i: (0, 0)),
            pl.BlockSpec((heads, cdim, cdim), lambda i: (0, 0, 0)),
            pl.BlockSpec(attn_out_b.shape, lambda i: (0, 0)),
            pl.BlockSpec(bil_w.shape, lambda i: (0, 0)),
            pl.BlockSpec(bil_b.shape, lambda i: (0, 0)),
            pl.BlockSpec((1, 1, 128), lambda i: (i, 0, 0)),
            pl.BlockSpec((zdim, 2 * cdim), lambda i: (0, 0)),
            pl.BlockSpec((1, 2 * cdim), lambda i: (0, 0)),
            pl.BlockSpec((cdim, cdim), lambda i: (0, 0)),
            pl.BlockSpec(mlp_out_b.shape, lambda i: (0, 0)),
        ],
        out_specs=pl.BlockSpec((1, t, cdim), lambda i: (i, 0, 0)),
        compiler_params=pltpu.CompilerParams(
            dimension_semantics=("parallel",),
            vmem_limit_bytes=100 * 1024 * 1024),
    )(x3, qkv_w.astype(bf), qkv_b, norm_mask, qs, qc, ks, kc, inj, m1, m5,
      attn_out_w.reshape(heads, cdim, cdim).astype(bf), attn_out_b,
      bil_w.astype(bf), bil_b, ref_ps, w1.astype(bf), b1,
      mlp_out_w.astype(bf), mlp_out_b)

    return out.reshape(b, t, c_h, MV)
